# baseline (speedup 1.0000x reference)
import functools

import jax
import jax.numpy as jnp
from jax import lax
from jax.experimental import pallas as pl
from jax.experimental.pallas import tpu as pltpu
from jax.experimental.pallas import tpu_sc as plsc

BATCH = 16384
N_OUT = 4
NW = 32
BPW = BATCH // NW
EPW = BPW * N_OUT
L = 16

_mesh = plsc.VectorSubcoreMesh(core_axis_name="c", subcore_axis_name="s")


@functools.partial(
    pl.kernel,
    mesh=_mesh,
    out_type=jax.ShapeDtypeStruct((BATCH * N_OUT,), jnp.float32),
    compiler_params=pltpu.CompilerParams(needs_layout_passes=False),
    scratch_types=[
        pltpu.VMEM((BPW,), jnp.int32),
        pltpu.VMEM((BPW,), jnp.int32),
        pltpu.VMEM((BPW,), jnp.int32),
        pltpu.VMEM((BPW,), jnp.int32),
        pltpu.VMEM((BPW,), jnp.int32),
        pltpu.VMEM((BPW,), jnp.int32),
        pltpu.VMEM((BPW,), jnp.float32),
        pltpu.VMEM((BPW,), jnp.float32),
        pltpu.VMEM((BPW,), jnp.float32),
        pltpu.VMEM((BPW,), jnp.float32),
        pltpu.VMEM((EPW,), jnp.float32),
        pltpu.SemaphoreType.DMA,
        pltpu.SemaphoreType.DMA,
        pltpu.SemaphoreType.DMA,
        pltpu.SemaphoreType.DMA,
    ],
)
def _nnlm_sc(x0_hbm, x1_hbm, cflat_hbm, out_hbm,
             x0_v, x1_v, ia_v, ib_v, ic_v, id_v,
             ca_v, cb_v, cc_v, cd_v, out_v,
             sem_a, sem_b, sem_c, sem_d):
    wid = lax.axis_index("s") * 2 + lax.axis_index("c")
    base = wid * BPW

    pltpu.sync_copy(x0_hbm.at[pl.ds(base, BPW)], x0_v)
    pltpu.sync_copy(x1_hbm.at[pl.ds(base, BPW)], x1_v)

    def expand(i, carry):
        sl = pl.ds(i * L, L)
        a = x0_v[sl] * 2
        ia_v[sl] = a
        ib_v[sl] = a + 1
        c = x1_v[sl] * 2
        ic_v[sl] = c
        id_v[sl] = c + 1
        return carry

    lax.fori_loop(0, BPW // L, expand, 0)

    cp_a = pltpu.async_copy(cflat_hbm.at[ia_v], ca_v, sem_a)
    cp_b = pltpu.async_copy(cflat_hbm.at[ib_v], cb_v, sem_b)
    cp_c = pltpu.async_copy(cflat_hbm.at[ic_v], cc_v, sem_c)
    cp_d = pltpu.async_copy(cflat_hbm.at[id_v], cd_v, sem_d)
    cp_a.wait()
    cp_b.wait()
    cp_c.wait()
    cp_d.wait()

    lanes = lax.iota(jnp.int32, L)

    def ftanh_exp(v):
        return jnp.exp(1.0 - 2.0 / (jnp.exp(2.0 * v) + 1.0))

    def step(i, carry):
        sl = pl.ds(i * L, L)
        ea = ftanh_exp(ca_v[sl])
        eb = ftanh_exp(cb_v[sl])
        ec = ftanh_exp(cc_v[sl])
        ed = ftanh_exp(cd_v[sl])
        r = 1.0 / ((ea + eb) + (ec + ed))
        pos = (i * L + lanes) * N_OUT
        plsc.store_scatter(out_v, [pos], ea * r)
        plsc.store_scatter(out_v, [pos + 1], eb * r)
        plsc.store_scatter(out_v, [pos + 2], ec * r)
        plsc.store_scatter(out_v, [pos + 3], ed * r)
        return carry

    lax.fori_loop(0, BPW // L, step, 0)
    pltpu.sync_copy(out_v, out_hbm.at[pl.ds(wid * EPW, EPW)])


def kernel(x, C):
    out_flat = _nnlm_sc(x[:, 0], x[:, 1], C.reshape(-1))
    return out_flat.reshape(BATCH, N_OUT)
